# Initial kernel scaffold; baseline (speedup 1.0000x reference)
import jax
import jax.numpy as jnp
from jax.experimental import pallas as pl


def kernel(hidden_states, router_weight, router_bias, gate_up_proj, gate_up_bias, down_proj, down_bias):
    raise NotImplementedError("write your pallas kernel here")


# speedup vs baseline: 1.6359x; 1.6359x over previous
import jax
import jax.numpy as jnp
from jax.experimental import pallas as pl
from jax.experimental.pallas import tpu as pltpu

E = 8
K = 2
H = 768
INTER = 768
ALPHA = 1.702
LIMIT = 7.0

TB = 512


def _moe_body(xf_ref, xb_ref, wr_ref, rb_ref, wgu_ref, bgu_ref, wd_ref, bd_ref,
              out_ref, w_scr):
    e = pl.program_id(1)

    @pl.when(e == 0)
    def _router():
        logits = jax.lax.dot_general(
            xf_ref[...], wr_ref[...], (((1,), (1,)), ((), ())),
            preferred_element_type=jnp.float32,
            precision=jax.lax.Precision.HIGHEST) + rb_ref[...]
        m = jnp.max(logits, axis=-1, keepdims=True)
        ex = jnp.exp(logits - m)
        p = ex / jnp.sum(ex, axis=-1, keepdims=True)
        lanes = jax.lax.broadcasted_iota(jnp.int32, (TB, E), 1)
        a1 = jnp.argmax(p, axis=-1)
        oh1 = lanes == a1[:, None]
        p2 = jnp.where(oh1, -jnp.inf, p)
        a2 = jnp.argmax(p2, axis=-1)
        oh2 = lanes == a2[:, None]
        w_scr[...] = p * (oh1 | oh2).astype(jnp.float32)
        out_ref[...] = jnp.zeros_like(out_ref)

    xb = xb_ref[...]
    gu = jnp.dot(xb, wgu_ref[0], preferred_element_type=jnp.float32) + bgu_ref[0, 0][None, :]
    gate = jnp.minimum(gu[:, :INTER], LIMIT)
    up = jnp.clip(gu[:, INTER:], -LIMIT, LIMIT)
    glu = gate * jax.nn.sigmoid(gate * ALPHA)
    act = ((up + 1.0) * glu).astype(jnp.bfloat16)
    dn = jnp.dot(act, wd_ref[0], preferred_element_type=jnp.float32) + bd_ref[0, 0][None, :]
    oh_e = (jax.lax.broadcasted_iota(jnp.int32, (E, 1), 0) == e).astype(jnp.float32)
    wcol = jnp.dot(w_scr[...], oh_e, preferred_element_type=jnp.float32)
    out_ref[...] += wcol * dn


def kernel(hidden_states, router_weight, router_bias, gate_up_proj, gate_up_bias,
           down_proj, down_bias):
    B, S, _ = hidden_states.shape
    xf = hidden_states.reshape(B * S, H)
    xb = xf.astype(jnp.bfloat16)
    wgu = gate_up_proj.astype(jnp.bfloat16)
    wd = down_proj.astype(jnp.bfloat16)
    rb = router_bias.reshape(1, E)
    bgu = gate_up_bias.reshape(E, 1, 2 * INTER)
    bd = down_bias.reshape(E, 1, H)

    grid = (B * S // TB, E)
    out = pl.pallas_call(
        _moe_body,
        grid=grid,
        in_specs=[
            pl.BlockSpec((TB, H), lambda t, e: (t, 0)),
            pl.BlockSpec((TB, H), lambda t, e: (t, 0)),
            pl.BlockSpec((E, H), lambda t, e: (0, 0)),
            pl.BlockSpec((1, E), lambda t, e: (0, 0)),
            pl.BlockSpec((1, H, 2 * INTER), lambda t, e: (e, 0, 0)),
            pl.BlockSpec((1, 1, 2 * INTER), lambda t, e: (e, 0, 0)),
            pl.BlockSpec((1, INTER, H), lambda t, e: (e, 0, 0)),
            pl.BlockSpec((1, 1, H), lambda t, e: (e, 0, 0)),
        ],
        out_specs=pl.BlockSpec((TB, H), lambda t, e: (t, 0)),
        out_shape=jax.ShapeDtypeStruct((B * S, H), jnp.float32),
        scratch_shapes=[pltpu.VMEM((TB, E), jnp.float32)],
    )(xf, xb, router_weight, rb, wgu, bgu, wd, bd)
    return out.reshape(B, S, H)


# speedup vs baseline: 1.7138x; 1.0476x over previous
import jax
import jax.numpy as jnp
from jax.experimental import pallas as pl
from jax.experimental.pallas import tpu as pltpu

E = 8
K = 2
H = 768
INTER = 768
ALPHA = 1.702
LIMIT = 7.0

TB = 512


def _moe_body(xb_ref, wr_ref, rb_ref, wgu_ref, bgu_ref, wd_ref, bd_ref,
              out_ref, w_scr):
    e = pl.program_id(1)

    @pl.when(e == 0)
    def _router():
        logits = jax.lax.dot_general(
            xb_ref[...], wr_ref[...], (((1,), (1,)), ((), ())),
            preferred_element_type=jnp.float32) + rb_ref[...]
        m = jnp.max(logits, axis=-1, keepdims=True)
        ex = jnp.exp(logits - m)
        p = ex / jnp.sum(ex, axis=-1, keepdims=True)
        lanes = jax.lax.broadcasted_iota(jnp.int32, (TB, E), 1)
        a1 = jnp.argmax(p, axis=-1)
        oh1 = lanes == a1[:, None]
        p2 = jnp.where(oh1, -jnp.inf, p)
        a2 = jnp.argmax(p2, axis=-1)
        oh2 = lanes == a2[:, None]
        w_scr[...] = p * (oh1 | oh2).astype(jnp.float32)
        out_ref[...] = jnp.zeros_like(out_ref)

    xb = xb_ref[...]
    gu = jnp.dot(xb, wgu_ref[0], preferred_element_type=jnp.float32) + bgu_ref[0, 0][None, :]
    gate = jnp.minimum(gu[:, :INTER], LIMIT)
    up = jnp.clip(gu[:, INTER:], -LIMIT, LIMIT)
    glu = gate * jax.nn.sigmoid(gate * ALPHA)
    act = ((up + 1.0) * glu).astype(jnp.bfloat16)
    dn = jnp.dot(act, wd_ref[0], preferred_element_type=jnp.float32) + bd_ref[0, 0][None, :]
    lane_e = jax.lax.broadcasted_iota(jnp.int32, (TB, E), 1)
    wcol = jnp.sum(jnp.where(lane_e == e, w_scr[...], 0.0), axis=1, keepdims=True)
    out_ref[...] += wcol * dn


def kernel(hidden_states, router_weight, router_bias, gate_up_proj, gate_up_bias,
           down_proj, down_bias):
    B, S, _ = hidden_states.shape
    xb = hidden_states.reshape(B * S, H).astype(jnp.bfloat16)
    wgu = gate_up_proj.astype(jnp.bfloat16)
    wd = down_proj.astype(jnp.bfloat16)
    wr = router_weight.astype(jnp.bfloat16)
    rb = router_bias.reshape(1, E)
    bgu = gate_up_bias.reshape(E, 1, 2 * INTER)
    bd = down_bias.reshape(E, 1, H)

    grid = (B * S // TB, E)
    out = pl.pallas_call(
        _moe_body,
        grid=grid,
        in_specs=[
            pl.BlockSpec((TB, H), lambda t, e: (t, 0)),
            pl.BlockSpec((E, H), lambda t, e: (0, 0)),
            pl.BlockSpec((1, E), lambda t, e: (0, 0)),
            pl.BlockSpec((1, H, 2 * INTER), lambda t, e: (e, 0, 0)),
            pl.BlockSpec((1, 1, 2 * INTER), lambda t, e: (e, 0, 0)),
            pl.BlockSpec((1, INTER, H), lambda t, e: (e, 0, 0)),
            pl.BlockSpec((1, 1, H), lambda t, e: (e, 0, 0)),
        ],
        out_specs=pl.BlockSpec((TB, H), lambda t, e: (t, 0)),
        out_shape=jax.ShapeDtypeStruct((B * S, H), jnp.float32),
        scratch_shapes=[pltpu.VMEM((TB, E), jnp.float32)],
    )(xb, wr, rb, wgu, bgu, wd, bd)
    return out.reshape(B, S, H)
